# baseline (speedup 1.0000x reference)
import jax
import jax.numpy as jnp
import numpy as np
from jax.experimental import pallas as pl
from jax.experimental.pallas import tpu as pltpu

B = 128
A = 100000
HC = 2048
BC = 2 * HC
K = (A + BC - 1) // BC

LOGP = np.float32(np.log(np.float64(np.float32(1.0 / A))))

_KS1 = np.uint32(42)
_KS2 = np.uint32(42 ^ 0x1BD11BDA)
_ROT_A = (13, 15, 26, 6)
_ROT_B = (17, 29, 16, 24)


def _rounds(x0, x1, rots):
    for d in rots:
        x0 = x0 + x1
        x1 = ((x1 << np.uint32(d)) | (x1 >> np.uint32(32 - d))) ^ x0
    return x0, x1


def _threefry_bits(x1):
    x0 = x1
    x1 = ((x1 << np.uint32(13)) | (x1 >> np.uint32(19))) ^ x0
    x0, x1 = _rounds(x0, x1, _ROT_A[1:])
    x0, x1 = x0 + _KS1, x1 + _KS2 + np.uint32(1)
    x0, x1 = _rounds(x0, x1, _ROT_B)
    x0, x1 = x0 + _KS2, x1 + np.uint32(2)
    x0, x1 = _rounds(x0, x1, _ROT_A)
    x0, x1 = x0, x1 + _KS1 + np.uint32(3)
    x0, x1 = _rounds(x0, x1, _ROT_B)
    x0, x1 = x0 + _KS1, x1 + _KS2 + np.uint32(4)
    x0, x1 = _rounds(x0, x1, _ROT_A)
    x0, x1 = x0 + _KS2, x1 + np.uint32(5)
    return x0 ^ x1


def _half(k, h):
    row = jax.lax.broadcasted_iota(jnp.int32, (B, HC), 0)
    colin = jax.lax.broadcasted_iota(jnp.int32, (B, HC), 1)
    col = jnp.minimum(colin + (k * BC + h * HC), A - 1)
    lin = (row * A + col).astype(jnp.uint32)
    bits = _threefry_bits(lin + _KS1)
    m = (bits >> np.uint32(9)).astype(jnp.int32)

    bmax = jnp.max(m, axis=1, keepdims=True)
    cand = jnp.where(m == bmax, col, jnp.int32(2**31 - 1))
    bidx = jnp.min(cand, axis=1, keepdims=True)
    return bmax, bidx


_C0 = 49152
_CT = A - 2 * _C0


def _sample_kernel(actions_ref, logprob_ref, bv_ref, bi_ref, cbuf, tbuf,
                   sems):
    k = pl.program_id(0)

    @pl.when(k == 0)
    def _fill_bufs():
        cbuf[...] = jnp.full((B, _C0), LOGP, dtype=jnp.float32)
        tbuf[...] = jnp.full((B, _CT), LOGP, dtype=jnp.float32)

    @pl.when(k == 1)
    def _dma0():
        pltpu.make_async_copy(
            cbuf, logprob_ref.at[:, pl.ds(0, _C0)], sems.at[0]).start()

    @pl.when(k == 9)
    def _dma1():
        pltpu.make_async_copy(
            cbuf, logprob_ref.at[:, pl.ds(_C0, _C0)], sems.at[1]).start()

    @pl.when(k == 17)
    def _dma2():
        pltpu.make_async_copy(
            tbuf, logprob_ref.at[:, pl.ds(2 * _C0, _CT)], sems.at[2]).start()

    bmax0, bidx0 = _half(k, 0)
    bmax1, bidx1 = _half(k, 1)
    bidx = jnp.where(bmax0 >= bmax1, bidx0, bidx1)
    bmax = jnp.maximum(bmax0, bmax1)

    @pl.when(k == 0)
    def _init():
        bv_ref[...] = bmax
        bi_ref[...] = bidx

    @pl.when(k > 0)
    def _combine():
        better = bmax > bv_ref[...]
        bi_ref[...] = jnp.where(better, bidx, bi_ref[...])
        bv_ref[...] = jnp.maximum(bmax, bv_ref[...])

    @pl.when(k == K - 1)
    def _emit():
        actions_ref[...] = bi_ref[...]
        pltpu.make_async_copy(
            cbuf, logprob_ref.at[:, pl.ds(0, _C0)], sems.at[0]).wait()
        pltpu.make_async_copy(
            cbuf, logprob_ref.at[:, pl.ds(_C0, _C0)], sems.at[1]).wait()
        pltpu.make_async_copy(
            tbuf, logprob_ref.at[:, pl.ds(2 * _C0, _CT)], sems.at[2]).wait()


@jax.jit
def _run():
    actions2d, logprob = pl.pallas_call(
        _sample_kernel,
        grid=(K,),
        out_specs=[
            pl.BlockSpec((B, 1), lambda k: (0, 0)),
            pl.BlockSpec(memory_space=pl.ANY),
        ],
        out_shape=[
            jax.ShapeDtypeStruct((B, 1), jnp.int32),
            jax.ShapeDtypeStruct((B, A), jnp.float32),
        ],
        scratch_shapes=[
            pltpu.VMEM((B, 1), jnp.int32),
            pltpu.VMEM((B, 1), jnp.int32),
            pltpu.VMEM((B, _C0), jnp.float32),
            pltpu.VMEM((B, _CT), jnp.float32),
            pltpu.SemaphoreType.DMA((3,)),
        ],
    )()
    return actions2d.reshape(B), logprob


def kernel(state):
    del state
    return _run()


# speedup vs baseline: 1.0007x; 1.0007x over previous
import jax
import jax.numpy as jnp
import numpy as np
from jax.experimental import pallas as pl
from jax.experimental.pallas import tpu as pltpu

B = 128
A = 100000
HC = 2048
BC = 2 * HC
K = (A + BC - 1) // BC

LOGP = np.float32(np.log(np.float64(np.float32(1.0 / A))))

_KS1 = np.uint32(42)
_KS2 = np.uint32(42 ^ 0x1BD11BDA)
_ROT_A = (13, 15, 26, 6)
_ROT_B = (17, 29, 16, 24)


def _rounds(x0, x1, rots):
    for d in rots:
        x0 = x0 + x1
        x1 = ((x1 << np.uint32(d)) | (x1 >> np.uint32(32 - d))) ^ x0
    return x0, x1


def _threefry_bits(x1):
    x0 = x1
    x1 = ((x1 << np.uint32(13)) | (x1 >> np.uint32(19))) ^ x0
    x0, x1 = _rounds(x0, x1, _ROT_A[1:])
    x0, x1 = x0 + _KS1, x1 + _KS2 + np.uint32(1)
    x0, x1 = _rounds(x0, x1, _ROT_B)
    x0, x1 = x0 + _KS2, x1 + np.uint32(2)
    x0, x1 = _rounds(x0, x1, _ROT_A)
    x0, x1 = x0, x1 + _KS1 + np.uint32(3)
    x0, x1 = _rounds(x0, x1, _ROT_B)
    x0, x1 = x0 + _KS1, x1 + _KS2 + np.uint32(4)
    x0, x1 = _rounds(x0, x1, _ROT_A)
    x0, x1 = x0 + _KS2, x1 + np.uint32(5)
    return x0 ^ x1


def _half(k, h):
    row = jax.lax.broadcasted_iota(jnp.int32, (B, HC), 0)
    colin = jax.lax.broadcasted_iota(jnp.int32, (B, HC), 1)
    col = jnp.minimum(colin + (k * BC + h * HC), A - 1)
    lin = (row * A + col).astype(jnp.uint32)
    bits = _threefry_bits(lin + _KS1)
    m = (bits >> np.uint32(9)).astype(jnp.int32)

    bmax = jnp.max(m, axis=1, keepdims=True)
    cand = jnp.where(m == bmax, col, jnp.int32(2**31 - 1))
    bidx = jnp.min(cand, axis=1, keepdims=True)
    return bmax, bidx


_C0 = 49152
_CT = A - 2 * _C0


def _sample_kernel(actions_ref, logprob_ref, bv_ref, bi_ref, cbuf, tbuf,
                   sems):
    k = pl.program_id(0)

    @pl.when(k == 0)
    def _fill_bufs():
        cbuf[...] = jnp.full((B, _C0), LOGP, dtype=jnp.float32)
        tbuf[...] = jnp.full((B, _CT), LOGP, dtype=jnp.float32)

    @pl.when(k == 1)
    def _dma0():
        pltpu.make_async_copy(
            cbuf, logprob_ref.at[:, pl.ds(0, _C0)], sems.at[0]).start()

    @pl.when(k == 9)
    def _dma1():
        pltpu.make_async_copy(
            cbuf, logprob_ref.at[:, pl.ds(_C0, _C0)], sems.at[1]).start()

    @pl.when(k == 17)
    def _dma2():
        pltpu.make_async_copy(
            tbuf, logprob_ref.at[:, pl.ds(2 * _C0, _CT)], sems.at[2]).start()

    bmax0, bidx0 = _half(k, 0)
    bmax1, bidx1 = _half(k, 1)
    bidx = jnp.where(bmax0 >= bmax1, bidx0, bidx1)
    bmax = jnp.maximum(bmax0, bmax1)

    @pl.when(k == 0)
    def _init():
        bv_ref[...] = bmax
        bi_ref[...] = bidx

    @pl.when(k > 0)
    def _combine():
        better = bmax > bv_ref[...]
        bi_ref[...] = jnp.where(better, bidx, bi_ref[...])
        bv_ref[...] = jnp.maximum(bmax, bv_ref[...])

    @pl.when(k == K - 1)
    def _emit():
        cp = pltpu.make_async_copy(bi_ref, actions_ref, sems.at[3])
        cp.start()
        cp.wait()
        pltpu.make_async_copy(
            cbuf, logprob_ref.at[:, pl.ds(0, _C0)], sems.at[0]).wait()
        pltpu.make_async_copy(
            cbuf, logprob_ref.at[:, pl.ds(_C0, _C0)], sems.at[1]).wait()
        pltpu.make_async_copy(
            tbuf, logprob_ref.at[:, pl.ds(2 * _C0, _CT)], sems.at[2]).wait()


@jax.jit
def _run():
    actions2d, logprob = pl.pallas_call(
        _sample_kernel,
        grid=(K,),
        out_specs=[
            pl.BlockSpec(memory_space=pl.ANY),
            pl.BlockSpec(memory_space=pl.ANY),
        ],
        out_shape=[
            jax.ShapeDtypeStruct((B, 1), jnp.int32),
            jax.ShapeDtypeStruct((B, A), jnp.float32),
        ],
        scratch_shapes=[
            pltpu.VMEM((B, 1), jnp.int32),
            pltpu.VMEM((B, 1), jnp.int32),
            pltpu.VMEM((B, _C0), jnp.float32),
            pltpu.VMEM((B, _CT), jnp.float32),
            pltpu.SemaphoreType.DMA((4,)),
        ],
    )()
    return actions2d.reshape(B), logprob


def kernel(state):
    del state
    return _run()


# speedup vs baseline: 1.0038x; 1.0030x over previous
import jax
import jax.numpy as jnp
import numpy as np
from jax.experimental import pallas as pl
from jax.experimental.pallas import tpu as pltpu

B = 128
A = 100000
HC = 2048
BC = 2 * HC
K = (A + BC - 1) // BC

LOGP = np.float32(np.log(np.float64(np.float32(1.0 / A))))

_KS1 = np.uint32(42)
_KS2 = np.uint32(42 ^ 0x1BD11BDA)
_ROT_A = (13, 15, 26, 6)
_ROT_B = (17, 29, 16, 24)


def _rounds(x0, x1, rots):
    for d in rots:
        x0 = x0 + x1
        x1 = ((x1 << np.uint32(d)) | (x1 >> np.uint32(32 - d))) ^ x0
    return x0, x1


def _threefry_bits(x1):
    x0 = x1
    x1 = ((x1 << np.uint32(13)) | (x1 >> np.uint32(19))) ^ x0
    x0, x1 = _rounds(x0, x1, _ROT_A[1:])
    x0, x1 = x0 + _KS1, x1 + _KS2 + np.uint32(1)
    x0, x1 = _rounds(x0, x1, _ROT_B)
    x0, x1 = x0 + _KS2, x1 + np.uint32(2)
    x0, x1 = _rounds(x0, x1, _ROT_A)
    x0, x1 = x0, x1 + _KS1 + np.uint32(3)
    x0, x1 = _rounds(x0, x1, _ROT_B)
    x0, x1 = x0 + _KS1, x1 + _KS2 + np.uint32(4)
    x0, x1 = _rounds(x0, x1, _ROT_A)
    x0, x1 = x0 + _KS2, x1 + np.uint32(5)
    return x0 ^ x1


def _half(k, h):
    row = jax.lax.broadcasted_iota(jnp.int32, (B, HC), 0)
    colin = jax.lax.broadcasted_iota(jnp.int32, (B, HC), 1)
    col = jnp.minimum(colin + (k * BC + h * HC), A - 1)
    lin = (row * A + col).astype(jnp.uint32)
    bits = _threefry_bits(lin + _KS1)
    m = (bits >> np.uint32(9)).astype(jnp.int32)

    bmax = jnp.max(m, axis=1, keepdims=True)
    cand = jnp.where(m == bmax, col, jnp.int32(2**31 - 1))
    bidx = jnp.min(cand, axis=1, keepdims=True)
    return bmax, bidx


_NCH = A // BC
_CT = A - _NCH * BC


def _chunk_copy(logprob_ref, cbuf, sems, c):
    return pltpu.make_async_copy(
        cbuf, logprob_ref.at[:, pl.ds(c * BC, BC)], sems.at[c % 4])


def _sample_kernel(actions_ref, logprob_ref, bv_ref, bi_ref, cbuf, tbuf,
                   sems):
    k = pl.program_id(0)

    @pl.when(k == 0)
    def _fill_bufs():
        cbuf[...] = jnp.full((B, BC), LOGP, dtype=jnp.float32)
        tbuf[...] = jnp.full((B, _CT), LOGP, dtype=jnp.float32)

    @pl.when((k >= 4) & (k < _NCH + 4))
    def _drain():
        _chunk_copy(logprob_ref, cbuf, sems, k - 4).wait()

    @pl.when(k < _NCH)
    def _start():
        _chunk_copy(logprob_ref, cbuf, sems, k).start()

    bmax0, bidx0 = _half(k, 0)
    bmax1, bidx1 = _half(k, 1)
    bidx = jnp.where(bmax0 >= bmax1, bidx0, bidx1)
    bmax = jnp.maximum(bmax0, bmax1)

    @pl.when(k == 0)
    def _init():
        bv_ref[...] = bmax
        bi_ref[...] = bidx

    @pl.when(k > 0)
    def _combine():
        better = bmax > bv_ref[...]
        bi_ref[...] = jnp.where(better, bidx, bi_ref[...])
        bv_ref[...] = jnp.maximum(bmax, bv_ref[...])

    @pl.when(k == K - 1)
    def _emit():
        tail = pltpu.make_async_copy(
            tbuf, logprob_ref.at[:, pl.ds(_NCH * BC, _CT)], sems.at[4])
        tail.start()
        cp = pltpu.make_async_copy(bi_ref, actions_ref, sems.at[4])
        cp.start()
        for c in range(_NCH - 3, _NCH):
            _chunk_copy(logprob_ref, cbuf, sems, c).wait()
        tail.wait()
        cp.wait()


@jax.jit
def _run():
    actions2d, logprob = pl.pallas_call(
        _sample_kernel,
        grid=(K,),
        out_specs=[
            pl.BlockSpec(memory_space=pl.ANY),
            pl.BlockSpec(memory_space=pl.ANY),
        ],
        out_shape=[
            jax.ShapeDtypeStruct((B, 1), jnp.int32),
            jax.ShapeDtypeStruct((B, A), jnp.float32),
        ],
        scratch_shapes=[
            pltpu.VMEM((B, 1), jnp.int32),
            pltpu.VMEM((B, 1), jnp.int32),
            pltpu.VMEM((B, BC), jnp.float32),
            pltpu.VMEM((B, _CT), jnp.float32),
            pltpu.SemaphoreType.DMA((5,)),
        ],
    )()
    return actions2d.reshape(B), logprob


def kernel(state):
    del state
    return _run()
